# Initial kernel scaffold; baseline (speedup 1.0000x reference)
import jax
import jax.numpy as jnp
from jax.experimental import pallas as pl


def kernel(geometry_points, surface_points, volume_points, bc_values, W_geo, W_bc, W_loc0, W_loc1, W_locproj, W_pt, W_blocks1, W_blocks2, W_head_vol, b_head_vol, W_head_surf, b_head_surf):
    raise NotImplementedError("write your pallas kernel here")


# speedup vs baseline: 3.8929x; 3.8929x over previous
import functools
import math

import jax
import jax.numpy as jnp
from jax.experimental import pallas as pl

M_FREQ = 8
KS = (8, 32)
RS = (0.05, 0.25)
BIG = 3.4e38


def _fourier(q, freqs):
    parts = [q]
    for c in range(3):
        ang = q[:, c:c + 1] * freqs
        parts.append(jnp.sin(ang))
        parts.append(jnp.cos(ang))
    return jnp.concatenate(parts, axis=1)


def _ctx_kernel(freqs_ref, geo_ref, wgeo_ref, bc_ref, wbc_ref, out_ref):
    geo = geo_ref[...]
    ff = _fourier(geo, freqs_ref[...])
    emb = jnp.dot(ff, wgeo_ref[...], preferred_element_type=jnp.float32)
    ctx = jnp.mean(emb, axis=0, keepdims=True)
    bc = jnp.dot(bc_ref[...], wbc_ref[...], preferred_element_type=jnp.float32)
    ctx = ctx + jnp.mean(jax.nn.relu(bc), axis=0, keepdims=True)
    out_ref[...] = ctx


def _main_kernel(freqs_ref, q_ref, src_ref, srcT_ref, wl0_ref, wl1_ref,
                 wproj_ref, wpt_ref, ctx_ref, wb1_ref, wb2_ref, whv_ref,
                 bhv_ref, whs_ref, bhs_ref, outv_ref, outs_ref, *, ng):
    freqs = freqs_ref[...]
    q = q_ref[...]
    tq = q.shape[0]
    src = src_ref[...]
    srcT = srcT_ref[...]

    qn = jnp.sum(q * q, axis=1, keepdims=True)
    sn = jnp.sum(srcT * srcT, axis=0, keepdims=True)
    cross = jnp.dot(q, srcT, preferred_element_type=jnp.float32)
    d2 = jnp.maximum(qn + sn - 2.0 * cross, 0.0)

    iota = jax.lax.broadcasted_iota(jnp.int32, (tq, ng), 1)

    wl0 = wl0_ref[...]
    wl1 = wl1_ref[...]
    qenc0 = jnp.dot(q, wl0[:3], preferred_element_type=jnp.float32)
    qenc1 = jnp.dot(q, wl1[:3], preferred_element_type=jnp.float32)
    r0sq = RS[0] * RS[0]
    r1sq = RS[1] * RS[1]

    pooled0 = jnp.zeros_like(qenc0)
    pooled1 = jnp.zeros_like(qenc1)

    def body(i, carry):
        d, pooled0, pooled1 = carry
        m = jnp.min(d, axis=1, keepdims=True)
        cand = jnp.where(d == m, iota, ng)
        idx = jnp.min(cand, axis=1, keepdims=True)
        first = cand == idx
        d = jnp.where(first, BIG, d)
        coords = jnp.dot(first.astype(jnp.float32), src,
                         preferred_element_type=jnp.float32)
        dist = jnp.sqrt(m)
        enc1 = jax.nn.relu(
            jnp.dot(coords, wl1[:3], preferred_element_type=jnp.float32)
            - qenc1 + dist * wl1[3:4])
        pooled1 = jnp.maximum(pooled1, jnp.where(m <= r1sq, enc1, 0.0))
        enc0 = jax.nn.relu(
            jnp.dot(coords, wl0[:3], preferred_element_type=jnp.float32)
            - qenc0 + dist * wl0[3:4])
        in0 = jnp.logical_and(m <= r0sq, i < KS[0])
        pooled0 = jnp.maximum(pooled0, jnp.where(in0, enc0, 0.0))
        return d, pooled0, pooled1

    _, pooled0, pooled1 = jax.lax.fori_loop(
        0, KS[1], body, (d2, pooled0, pooled1))

    feat = jnp.concatenate([pooled0, pooled1], axis=1)
    bq = jnp.dot(feat, wproj_ref[...], preferred_element_type=jnp.float32)

    ff = _fourier(q, freqs)
    x = jnp.dot(ff, wpt_ref[...], preferred_element_type=jnp.float32)
    x = x + bq + ctx_ref[...]

    wb1 = wb1_ref[...]
    wb2 = wb2_ref[...]
    for l in range(wb1.shape[0]):
        h = jax.nn.relu(jnp.dot(x, wb1[l], preferred_element_type=jnp.float32))
        x = x + jnp.dot(h, wb2[l], preferred_element_type=jnp.float32)

    outv_ref[...] = jnp.dot(x, whv_ref[...],
                            preferred_element_type=jnp.float32) + bhv_ref[...]
    outs_ref[...] = jnp.dot(x, whs_ref[...],
                            preferred_element_type=jnp.float32) + bhs_ref[...]


def kernel(geometry_points, surface_points, volume_points, bc_values, W_geo,
           W_bc, W_loc0, W_loc1, W_locproj, W_pt, W_blocks1, W_blocks2,
           W_head_vol, b_head_vol, W_head_surf, b_head_surf):
    b, ng, _ = geometry_points.shape
    nv = volume_points.shape[1]
    ns = surface_points.shape[1]
    h = W_geo.shape[1]

    freqs = (2.0 ** jnp.arange(M_FREQ, dtype=jnp.float32))[None, :] * math.pi

    geo = geometry_points[0]
    srcT = geo.T
    bc = bc_values[0]

    ctx = pl.pallas_call(
        _ctx_kernel,
        out_shape=jax.ShapeDtypeStruct((1, h), jnp.float32),
    )(freqs, geo, W_geo, bc, W_bc)

    pts = jnp.concatenate([volume_points[0], surface_points[0]], axis=0)
    nq = nv + ns
    tile = 256
    grid = nq // tile

    full = lambda *shape: pl.BlockSpec(shape, lambda t: (0,) * len(shape))
    outv, outs = pl.pallas_call(
        functools.partial(_main_kernel, ng=ng),
        grid=(grid,),
        in_specs=[
            full(1, M_FREQ),
            pl.BlockSpec((tile, 3), lambda t: (t, 0)),
            full(ng, 3),
            full(3, ng),
            full(4, W_loc0.shape[1]),
            full(4, W_loc1.shape[1]),
            full(W_locproj.shape[0], h),
            full(W_pt.shape[0], h),
            full(1, h),
            full(*W_blocks1.shape),
            full(*W_blocks2.shape),
            full(h, W_head_vol.shape[1]),
            full(1, W_head_vol.shape[1]),
            full(h, W_head_surf.shape[1]),
            full(1, W_head_surf.shape[1]),
        ],
        out_specs=[
            pl.BlockSpec((tile, W_head_vol.shape[1]), lambda t: (t, 0)),
            pl.BlockSpec((tile, W_head_surf.shape[1]), lambda t: (t, 0)),
        ],
        out_shape=[
            jax.ShapeDtypeStruct((nq, W_head_vol.shape[1]), jnp.float32),
            jax.ShapeDtypeStruct((nq, W_head_surf.shape[1]), jnp.float32),
        ],
    )(freqs, pts, geo, srcT, W_loc0, W_loc1, W_locproj, W_pt, ctx, W_blocks1,
      W_blocks2, W_head_vol, b_head_vol[None, :], W_head_surf,
      b_head_surf[None, :])

    pred_vol = outv[:nv][None]
    pred_surf = outs[nv:][None]
    return (pred_vol, pred_surf)


# speedup vs baseline: 4.2462x; 1.0908x over previous
import functools
import math

import jax
import jax.numpy as jnp
from jax.experimental import pallas as pl

M_FREQ = 8
KS = (8, 32)
RS = (0.05, 0.25)
BIG = 3.4e38


def _fourier(q, freqs):
    parts = [q]
    for c in range(3):
        ang = q[:, c:c + 1] * freqs
        parts.append(jnp.sin(ang))
        parts.append(jnp.cos(ang))
    return jnp.concatenate(parts, axis=1)


def _ctx_kernel(freqs_ref, geo_ref, wgeo_ref, bc_ref, wbc_ref, out_ref):
    geo = geo_ref[...]
    ff = _fourier(geo, freqs_ref[...])
    emb = jnp.dot(ff, wgeo_ref[...], preferred_element_type=jnp.float32)
    ctx = jnp.mean(emb, axis=0, keepdims=True)
    bc = jnp.dot(bc_ref[...], wbc_ref[...], preferred_element_type=jnp.float32)
    ctx = ctx + jnp.mean(jax.nn.relu(bc), axis=0, keepdims=True)
    out_ref[...] = ctx


def _main_kernel(freqs_ref, q_ref, src_ref, srcT_ref, wl0_ref, wl1_ref,
                 wproj_ref, wpt_ref, ctx_ref, wb1_ref, wb2_ref, whv_ref,
                 bhv_ref, whs_ref, bhs_ref, outv_ref, outs_ref, *, ng):
    freqs = freqs_ref[...]
    q = q_ref[...]
    tq = q.shape[0]
    src = src_ref[...]
    srcT = srcT_ref[...]

    qn = jnp.sum(q * q, axis=1, keepdims=True)
    sn = jnp.sum(srcT * srcT, axis=0, keepdims=True)
    cross = jnp.dot(q, srcT, preferred_element_type=jnp.float32)
    d2 = jnp.maximum(qn + sn - 2.0 * cross, 0.0)

    iota = jax.lax.broadcasted_iota(jnp.int32, (tq, ng), 1)

    wl0 = wl0_ref[...]
    wl1 = wl1_ref[...]
    qenc0 = jnp.dot(q, wl0[:3], preferred_element_type=jnp.float32)
    qenc1 = jnp.dot(q, wl1[:3], preferred_element_type=jnp.float32)
    r0sq = RS[0] * RS[0]
    r1sq = RS[1] * RS[1]

    pooled0 = jnp.zeros_like(qenc0)
    pooled1 = jnp.zeros_like(qenc1)

    def extract(d):
        m = jnp.min(d, axis=1, keepdims=True)
        cand = jnp.where(d == m, iota, ng)
        idx = jnp.min(cand, axis=1, keepdims=True)
        first = cand == idx
        d = jnp.where(first, BIG, d)
        coords = jnp.dot(first.astype(jnp.float32), src,
                         preferred_element_type=jnp.float32)
        return d, coords, m

    def enc_update(pooled, coords, dist, m, wl, qenc, rsq):
        enc = jax.nn.relu(
            jnp.dot(coords, wl[:3], preferred_element_type=jnp.float32)
            - qenc + dist * wl[3:4])
        return jnp.maximum(pooled, jnp.where(m <= rsq, enc, 0.0))

    def body01(_, carry):
        d, pooled0, pooled1 = carry
        d, coords, m = extract(d)
        dist = jnp.sqrt(m)
        pooled1 = enc_update(pooled1, coords, dist, m, wl1, qenc1, r1sq)
        pooled0 = enc_update(pooled0, coords, dist, m, wl0, qenc0, r0sq)
        return d, pooled0, pooled1

    def body1(_, carry):
        d, pooled1 = carry
        d, coords, m = extract(d)
        dist = jnp.sqrt(m)
        pooled1 = enc_update(pooled1, coords, dist, m, wl1, qenc1, r1sq)
        return d, pooled1

    d, pooled0, pooled1 = jax.lax.fori_loop(
        0, KS[0], body01, (d2, pooled0, pooled1))
    _, pooled1 = jax.lax.fori_loop(
        KS[0], KS[1], body1, (d, pooled1))

    feat = jnp.concatenate([pooled0, pooled1], axis=1)
    bq = jnp.dot(feat, wproj_ref[...], preferred_element_type=jnp.float32)

    ff = _fourier(q, freqs)
    x = jnp.dot(ff, wpt_ref[...], preferred_element_type=jnp.float32)
    x = x + bq + ctx_ref[...]

    wb1 = wb1_ref[...]
    wb2 = wb2_ref[...]
    for l in range(wb1.shape[0]):
        h = jax.nn.relu(jnp.dot(x, wb1[l], preferred_element_type=jnp.float32))
        x = x + jnp.dot(h, wb2[l], preferred_element_type=jnp.float32)

    outv_ref[...] = jnp.dot(x, whv_ref[...],
                            preferred_element_type=jnp.float32) + bhv_ref[...]
    outs_ref[...] = jnp.dot(x, whs_ref[...],
                            preferred_element_type=jnp.float32) + bhs_ref[...]


def kernel(geometry_points, surface_points, volume_points, bc_values, W_geo,
           W_bc, W_loc0, W_loc1, W_locproj, W_pt, W_blocks1, W_blocks2,
           W_head_vol, b_head_vol, W_head_surf, b_head_surf):
    b, ng, _ = geometry_points.shape
    nv = volume_points.shape[1]
    ns = surface_points.shape[1]
    h = W_geo.shape[1]

    freqs = (2.0 ** jnp.arange(M_FREQ, dtype=jnp.float32))[None, :] * math.pi

    geo = geometry_points[0]
    srcT = geo.T
    bc = bc_values[0]

    ctx = pl.pallas_call(
        _ctx_kernel,
        out_shape=jax.ShapeDtypeStruct((1, h), jnp.float32),
    )(freqs, geo, W_geo, bc, W_bc)

    pts = jnp.concatenate([volume_points[0], surface_points[0]], axis=0)
    nq = nv + ns
    tile = next(t for t in (512, 256, 128, 64, 32, 16, 8) if nq % t == 0)
    grid = nq // tile

    full = lambda *shape: pl.BlockSpec(shape, lambda t: (0,) * len(shape))
    outv, outs = pl.pallas_call(
        functools.partial(_main_kernel, ng=ng),
        grid=(grid,),
        in_specs=[
            full(1, M_FREQ),
            pl.BlockSpec((tile, 3), lambda t: (t, 0)),
            full(ng, 3),
            full(3, ng),
            full(4, W_loc0.shape[1]),
            full(4, W_loc1.shape[1]),
            full(W_locproj.shape[0], h),
            full(W_pt.shape[0], h),
            full(1, h),
            full(*W_blocks1.shape),
            full(*W_blocks2.shape),
            full(h, W_head_vol.shape[1]),
            full(1, W_head_vol.shape[1]),
            full(h, W_head_surf.shape[1]),
            full(1, W_head_surf.shape[1]),
        ],
        out_specs=[
            pl.BlockSpec((tile, W_head_vol.shape[1]), lambda t: (t, 0)),
            pl.BlockSpec((tile, W_head_surf.shape[1]), lambda t: (t, 0)),
        ],
        out_shape=[
            jax.ShapeDtypeStruct((nq, W_head_vol.shape[1]), jnp.float32),
            jax.ShapeDtypeStruct((nq, W_head_surf.shape[1]), jnp.float32),
        ],
    )(freqs, pts, geo, srcT, W_loc0, W_loc1, W_locproj, W_pt, ctx, W_blocks1,
      W_blocks2, W_head_vol, b_head_vol[None, :], W_head_surf,
      b_head_surf[None, :])

    pred_vol = outv[:nv][None]
    pred_surf = outs[nv:][None]
    return (pred_vol, pred_surf)


# speedup vs baseline: 4.5256x; 1.0658x over previous
import functools
import math

import jax
import jax.numpy as jnp
from jax.experimental import pallas as pl

M_FREQ = 8
KS = (8, 32)
RS = (0.05, 0.25)
BIG = 3.4e38


def _split_fourier_weight(w):
    w_xyz = w[:3]
    w_sin = jnp.stack([w[3 + 2 * M_FREQ * c: 3 + 2 * M_FREQ * c + M_FREQ]
                       for c in range(3)])
    w_cos = jnp.stack([w[3 + 2 * M_FREQ * c + M_FREQ: 3 + 2 * M_FREQ * (c + 1)]
                       for c in range(3)])
    return w_xyz, w_sin, w_cos


def _fourier_matmul(q, freqs, w_xyz, w_sin, w_cos):
    out = jnp.dot(q, w_xyz, preferred_element_type=jnp.float32)
    for c in range(3):
        ang = q[:, c:c + 1] * freqs
        out = out + jnp.dot(jnp.sin(ang), w_sin[c],
                            preferred_element_type=jnp.float32)
        out = out + jnp.dot(jnp.cos(ang), w_cos[c],
                            preferred_element_type=jnp.float32)
    return out


def _ctx_kernel(freqs_ref, geo_ref, wgx_ref, wgs_ref, wgc_ref, bc_ref,
                wbc_ref, out_ref):
    geo = geo_ref[...]
    emb = _fourier_matmul(geo, freqs_ref[...], wgx_ref[...], wgs_ref[...],
                          wgc_ref[...])
    ctx = jnp.mean(emb, axis=0, keepdims=True)
    bc = jnp.dot(bc_ref[...], wbc_ref[...], preferred_element_type=jnp.float32)
    ctx = ctx + jnp.mean(jax.nn.relu(bc), axis=0, keepdims=True)
    out_ref[...] = ctx


def _main_kernel(freqs_ref, q_ref, src_ref, srcT_ref, wl0_ref, wl1_ref,
                 wproj0_ref, wproj1_ref, wptx_ref, wpts_ref, wptc_ref,
                 ctx_ref, wb1_ref, wb2_ref, whv_ref,
                 bhv_ref, whs_ref, bhs_ref, outv_ref, outs_ref, *, ng):
    q = q_ref[...]
    tq = q.shape[0]
    src = src_ref[...]
    srcT = srcT_ref[...]

    qn = jnp.sum(q * q, axis=1, keepdims=True)
    sn = jnp.sum(srcT * srcT, axis=0, keepdims=True)
    cross = jnp.dot(q, srcT, preferred_element_type=jnp.float32)
    d2 = jnp.maximum(qn + sn - 2.0 * cross, 0.0)

    iota = jax.lax.broadcasted_iota(jnp.int32, (tq, ng), 1)

    wl0 = wl0_ref[...]
    wl1 = wl1_ref[...]
    qenc0 = jnp.dot(q, wl0[:3], preferred_element_type=jnp.float32)
    qenc1 = jnp.dot(q, wl1[:3], preferred_element_type=jnp.float32)
    r0sq = RS[0] * RS[0]
    r1sq = RS[1] * RS[1]

    pooled0 = jnp.zeros_like(qenc0)
    pooled1 = jnp.zeros_like(qenc1)

    def extract(d):
        m = jnp.min(d, axis=1, keepdims=True)
        idx = jnp.argmin(d, axis=1, keepdims=True)
        first = iota == idx
        d = jnp.where(first, BIG, d)
        coords = jnp.dot(first.astype(jnp.float32), src,
                         preferred_element_type=jnp.float32)
        return d, coords, m

    def enc_update(pooled, coords, dist, m, wl, qenc, rsq):
        enc = jax.nn.relu(
            jnp.dot(coords, wl[:3], preferred_element_type=jnp.float32)
            - qenc + dist * wl[3:4])
        return jnp.maximum(pooled, jnp.where(m <= rsq, enc, 0.0))

    def body01(_, carry):
        d, pooled0, pooled1 = carry
        d, coords, m = extract(d)
        dist = jnp.sqrt(m)
        pooled1 = enc_update(pooled1, coords, dist, m, wl1, qenc1, r1sq)
        pooled0 = enc_update(pooled0, coords, dist, m, wl0, qenc0, r0sq)
        return d, pooled0, pooled1

    def body1(_, carry):
        d, pooled1 = carry
        d, coords, m = extract(d)
        dist = jnp.sqrt(m)
        pooled1 = enc_update(pooled1, coords, dist, m, wl1, qenc1, r1sq)
        return d, pooled1

    d, pooled0, pooled1 = jax.lax.fori_loop(
        0, KS[0], body01, (d2, pooled0, pooled1))
    _, pooled1 = jax.lax.fori_loop(
        KS[0], KS[1], body1, (d, pooled1))

    bq = jnp.dot(pooled0, wproj0_ref[...], preferred_element_type=jnp.float32)
    bq = bq + jnp.dot(pooled1, wproj1_ref[...],
                      preferred_element_type=jnp.float32)

    x = _fourier_matmul(q, freqs_ref[...], wptx_ref[...], wpts_ref[...],
                        wptc_ref[...])
    x = x + bq + ctx_ref[...]

    wb1 = wb1_ref[...]
    wb2 = wb2_ref[...]
    for l in range(wb1.shape[0]):
        h = jax.nn.relu(jnp.dot(x, wb1[l], preferred_element_type=jnp.float32))
        x = x + jnp.dot(h, wb2[l], preferred_element_type=jnp.float32)

    outv_ref[...] = jnp.dot(x, whv_ref[...],
                            preferred_element_type=jnp.float32) + bhv_ref[...]
    outs_ref[...] = jnp.dot(x, whs_ref[...],
                            preferred_element_type=jnp.float32) + bhs_ref[...]


def kernel(geometry_points, surface_points, volume_points, bc_values, W_geo,
           W_bc, W_loc0, W_loc1, W_locproj, W_pt, W_blocks1, W_blocks2,
           W_head_vol, b_head_vol, W_head_surf, b_head_surf):
    b, ng, _ = geometry_points.shape
    nv = volume_points.shape[1]
    ns = surface_points.shape[1]
    h = W_geo.shape[1]

    freqs = (2.0 ** jnp.arange(M_FREQ, dtype=jnp.float32))[None, :] * math.pi
    wgx, wgs, wgc = _split_fourier_weight(W_geo)
    wptx, wpts, wptc = _split_fourier_weight(W_pt)

    geo = geometry_points[0]
    srcT = geo.T
    bc = bc_values[0]

    ctx = pl.pallas_call(
        _ctx_kernel,
        out_shape=jax.ShapeDtypeStruct((1, h), jnp.float32),
    )(freqs, geo, wgx, wgs, wgc, bc, W_bc)

    pts = jnp.concatenate([volume_points[0], surface_points[0]], axis=0)
    nq = nv + ns
    tile = next(t for t in (512, 256, 128, 64, 32, 16, 8) if nq % t == 0)
    grid = nq // tile

    full = lambda *shape: pl.BlockSpec(shape, lambda t: (0,) * len(shape))
    outv, outs = pl.pallas_call(
        functools.partial(_main_kernel, ng=ng),
        grid=(grid,),
        in_specs=[
            full(1, M_FREQ),
            pl.BlockSpec((tile, 3), lambda t: (t, 0)),
            full(ng, 3),
            full(3, ng),
            full(4, W_loc0.shape[1]),
            full(4, W_loc1.shape[1]),
            full(W_loc0.shape[1], h),
            full(W_loc1.shape[1], h),
            full(3, h),
            full(3, M_FREQ, h),
            full(3, M_FREQ, h),
            full(1, h),
            full(*W_blocks1.shape),
            full(*W_blocks2.shape),
            full(h, W_head_vol.shape[1]),
            full(1, W_head_vol.shape[1]),
            full(h, W_head_surf.shape[1]),
            full(1, W_head_surf.shape[1]),
        ],
        out_specs=[
            pl.BlockSpec((tile, W_head_vol.shape[1]), lambda t: (t, 0)),
            pl.BlockSpec((tile, W_head_surf.shape[1]), lambda t: (t, 0)),
        ],
        out_shape=[
            jax.ShapeDtypeStruct((nq, W_head_vol.shape[1]), jnp.float32),
            jax.ShapeDtypeStruct((nq, W_head_surf.shape[1]), jnp.float32),
        ],
    )(freqs, pts, geo, srcT, W_loc0, W_loc1, W_locproj[:W_loc0.shape[1]],
      W_locproj[W_loc0.shape[1]:], wptx, wpts, wptc, ctx, W_blocks1,
      W_blocks2, W_head_vol, b_head_vol[None, :], W_head_surf,
      b_head_surf[None, :])

    pred_vol = outv[:nv][None]
    pred_surf = outs[nv:][None]
    return (pred_vol, pred_surf)


# speedup vs baseline: 4.6031x; 1.0171x over previous
import functools
import math

import jax
import jax.numpy as jnp
from jax.experimental import pallas as pl

M_FREQ = 8
KS = (8, 32)
RS = (0.05, 0.25)
BIG = 3.4e38


def _split_fourier_weight(w):
    w_xyz = w[:3]
    w_sin = jnp.stack([w[3 + 2 * M_FREQ * c: 3 + 2 * M_FREQ * c + M_FREQ]
                       for c in range(3)])
    w_cos = jnp.stack([w[3 + 2 * M_FREQ * c + M_FREQ: 3 + 2 * M_FREQ * (c + 1)]
                       for c in range(3)])
    return w_xyz, w_sin, w_cos


def _fourier_matmul(q, freqs, w_xyz, w_sin, w_cos):
    out = jnp.dot(q, w_xyz, preferred_element_type=jnp.float32)
    for c in range(3):
        ang = q[:, c:c + 1] * freqs
        out = out + jnp.dot(jnp.sin(ang), w_sin[c],
                            preferred_element_type=jnp.float32)
        out = out + jnp.dot(jnp.cos(ang), w_cos[c],
                            preferred_element_type=jnp.float32)
    return out


def _ctx_kernel(freqs_ref, geo_ref, wgx_ref, wgs_ref, wgc_ref, bc_ref,
                wbc_ref, out_ref):
    geo = geo_ref[...]
    emb = _fourier_matmul(geo, freqs_ref[...], wgx_ref[...], wgs_ref[...],
                          wgc_ref[...])
    ctx = jnp.mean(emb, axis=0, keepdims=True)
    bc = jnp.dot(bc_ref[...], wbc_ref[...], preferred_element_type=jnp.float32)
    ctx = ctx + jnp.mean(jax.nn.relu(bc), axis=0, keepdims=True)
    out_ref[...] = ctx


def _main_kernel(freqs_ref, q_ref, src_ref, srcT_ref, wl0_ref, wl1_ref,
                 wproj0_ref, wproj1_ref, wptx_ref, wpts_ref, wptc_ref,
                 ctx_ref, wb1_ref, wb2_ref, whv_ref,
                 bhv_ref, whs_ref, bhs_ref, outv_ref, outs_ref, *, ng):
    q = q_ref[...]
    tq = q.shape[0]
    src = src_ref[...]
    srcT = srcT_ref[...]

    qn = jnp.sum(q * q, axis=1, keepdims=True)
    sn = jnp.sum(srcT * srcT, axis=0, keepdims=True)
    cross = jnp.dot(q, srcT, preferred_element_type=jnp.float32)
    d2 = jnp.maximum(qn + sn - 2.0 * cross, 0.0)

    wl0 = wl0_ref[...]
    wl1 = wl1_ref[...]
    qenc0 = jnp.dot(q, wl0[:3], preferred_element_type=jnp.float32)
    qenc1 = jnp.dot(q, wl1[:3], preferred_element_type=jnp.float32)
    r0sq = RS[0] * RS[0]
    r1sq = RS[1] * RS[1]

    pooled0 = jnp.zeros_like(qenc0)
    pooled1 = jnp.zeros_like(qenc1)

    def extract(d):
        m = jnp.min(d, axis=1, keepdims=True)
        idx = jnp.argmin(d, axis=1, keepdims=True)
        iota = jax.lax.broadcasted_iota(jnp.int32, d.shape, 1)
        first = iota == idx
        d = jnp.where(first, BIG, d)
        coords = jnp.dot(first.astype(jnp.float32), src,
                         preferred_element_type=jnp.float32)
        return d, coords, m

    def enc_update(pooled, coords, dist, m, wl, qenc, rsq):
        enc = jax.nn.relu(
            jnp.dot(coords, wl[:3], preferred_element_type=jnp.float32)
            - qenc + dist * wl[3:4])
        return jnp.maximum(pooled, jnp.where(m <= rsq, enc, 0.0))

    def body01(_, carry):
        d, pooled0, pooled1 = carry
        d, coords, m = extract(d)
        dist = jnp.sqrt(m)
        pooled1 = enc_update(pooled1, coords, dist, m, wl1, qenc1, r1sq)
        pooled0 = enc_update(pooled0, coords, dist, m, wl0, qenc0, r0sq)
        return d, pooled0, pooled1

    def body1(_, carry):
        d, pooled1 = carry
        d, coords, m = extract(d)
        dist = jnp.sqrt(m)
        pooled1 = enc_update(pooled1, coords, dist, m, wl1, qenc1, r1sq)
        return d, pooled1

    d, pooled0, pooled1 = jax.lax.fori_loop(
        0, KS[0], body01, (d2, pooled0, pooled1))
    _, pooled1 = jax.lax.fori_loop(
        KS[0], KS[1], body1, (d, pooled1))

    bq = jnp.dot(pooled0, wproj0_ref[...], preferred_element_type=jnp.float32)
    bq = bq + jnp.dot(pooled1, wproj1_ref[...],
                      preferred_element_type=jnp.float32)

    x = _fourier_matmul(q, freqs_ref[...], wptx_ref[...], wpts_ref[...],
                        wptc_ref[...])
    x = x + bq + ctx_ref[...]

    wb1 = wb1_ref[...]
    wb2 = wb2_ref[...]
    for l in range(wb1.shape[0]):
        h = jax.nn.relu(jnp.dot(x, wb1[l], preferred_element_type=jnp.float32))
        x = x + jnp.dot(h, wb2[l], preferred_element_type=jnp.float32)

    outv_ref[...] = jnp.dot(x, whv_ref[...],
                            preferred_element_type=jnp.float32) + bhv_ref[...]
    outs_ref[...] = jnp.dot(x, whs_ref[...],
                            preferred_element_type=jnp.float32) + bhs_ref[...]


def kernel(geometry_points, surface_points, volume_points, bc_values, W_geo,
           W_bc, W_loc0, W_loc1, W_locproj, W_pt, W_blocks1, W_blocks2,
           W_head_vol, b_head_vol, W_head_surf, b_head_surf):
    b, ng, _ = geometry_points.shape
    nv = volume_points.shape[1]
    ns = surface_points.shape[1]
    h = W_geo.shape[1]

    freqs = (2.0 ** jnp.arange(M_FREQ, dtype=jnp.float32))[None, :] * math.pi
    wgx, wgs, wgc = _split_fourier_weight(W_geo)
    wptx, wpts, wptc = _split_fourier_weight(W_pt)

    geo = geometry_points[0]
    srcT = geo.T
    bc = bc_values[0]

    ctx = pl.pallas_call(
        _ctx_kernel,
        out_shape=jax.ShapeDtypeStruct((1, h), jnp.float32),
    )(freqs, geo, wgx, wgs, wgc, bc, W_bc)

    pts = jnp.concatenate([volume_points[0], surface_points[0]], axis=0)
    nq = nv + ns
    tile = next(t for t in (1024, 512, 256, 128, 64, 32, 16, 8) if nq % t == 0)
    grid = nq // tile

    full = lambda *shape: pl.BlockSpec(shape, lambda t: (0,) * len(shape))
    outv, outs = pl.pallas_call(
        functools.partial(_main_kernel, ng=ng),
        grid=(grid,),
        in_specs=[
            full(1, M_FREQ),
            pl.BlockSpec((tile, 3), lambda t: (t, 0)),
            full(ng, 3),
            full(3, ng),
            full(4, W_loc0.shape[1]),
            full(4, W_loc1.shape[1]),
            full(W_loc0.shape[1], h),
            full(W_loc1.shape[1], h),
            full(3, h),
            full(3, M_FREQ, h),
            full(3, M_FREQ, h),
            full(1, h),
            full(*W_blocks1.shape),
            full(*W_blocks2.shape),
            full(h, W_head_vol.shape[1]),
            full(1, W_head_vol.shape[1]),
            full(h, W_head_surf.shape[1]),
            full(1, W_head_surf.shape[1]),
        ],
        out_specs=[
            pl.BlockSpec((tile, W_head_vol.shape[1]), lambda t: (t, 0)),
            pl.BlockSpec((tile, W_head_surf.shape[1]), lambda t: (t, 0)),
        ],
        out_shape=[
            jax.ShapeDtypeStruct((nq, W_head_vol.shape[1]), jnp.float32),
            jax.ShapeDtypeStruct((nq, W_head_surf.shape[1]), jnp.float32),
        ],
    )(freqs, pts, geo, srcT, W_loc0, W_loc1, W_locproj[:W_loc0.shape[1]],
      W_locproj[W_loc0.shape[1]:], wptx, wpts, wptc, ctx, W_blocks1,
      W_blocks2, W_head_vol, b_head_vol[None, :], W_head_surf,
      b_head_surf[None, :])

    pred_vol = outv[:nv][None]
    pred_surf = outs[nv:][None]
    return (pred_vol, pred_surf)
